# Initial kernel scaffold; baseline (speedup 1.0000x reference)
import jax
import jax.numpy as jnp
from jax.experimental import pallas as pl


def kernel(x, edge_index, batch, W1_0, b1_0, W2_0, b2_0, gamma_0, beta_0, W1_1, b1_1, W2_1, b2_1, gamma_1, beta_1, W1_2, b1_2, W2_2, b2_2, gamma_2, beta_2):
    raise NotImplementedError("write your pallas kernel here")


# speedup vs baseline: 4.6352x; 4.6352x over previous
import functools

import jax
import jax.numpy as jnp
from jax import lax
from jax.experimental import pallas as pl
from jax.experimental.pallas import tpu as pltpu
from jax.experimental.pallas import tpu_sc as plsc

N = 10000
E = 320000
D = 128
G = 128

NC = 2
NS = 16
NW = NC * NS

CHUNK = 80
EPW = E // NW
NCHUNK = EPW // CHUNK
ROWS_PER_SUB = 624
TAIL_START = NS * ROWS_PER_SUB
TAIL_ROWS = N - TAIL_START

_SC_MESH = plsc.VectorSubcoreMesh(core_axis_name="c", subcore_axis_name="s")


def _agg_body(src_hbm, dst_hbm, h_hbm, zero_hbm, out_hbm,
              src_v, dst_v, rows_v, acc_sh, sem):
    cid = lax.axis_index("c")
    sid = lax.axis_index("s")
    wid = sid * NC + cid
    r0 = sid * ROWS_PER_SUB
    pltpu.sync_copy(zero_hbm.at[pl.ds(r0, ROWS_PER_SUB)],
                    acc_sh.at[pl.ds(r0, ROWS_PER_SUB)])

    @pl.when(sid == NS - 1)
    def _():
        pltpu.sync_copy(zero_hbm.at[pl.ds(TAIL_START, TAIL_ROWS)],
                        acc_sh.at[pl.ds(TAIL_START, TAIL_ROWS)])

    plsc.subcore_barrier()

    def body(i, carry):
        base = wid * EPW + i * CHUNK
        pltpu.sync_copy(src_hbm.at[pl.ds(base, CHUNK)], src_v)
        pltpu.sync_copy(dst_hbm.at[pl.ds(base, CHUNK)], dst_v)
        pltpu.async_copy(h_hbm.at[src_v], rows_v, sem).wait()
        pltpu.sync_copy(rows_v, acc_sh.at[dst_v], add=True)
        return carry

    lax.fori_loop(0, NCHUNK, body, 0)
    plsc.subcore_barrier()
    pltpu.sync_copy(acc_sh.at[pl.ds(r0, ROWS_PER_SUB)],
                    out_hbm.at[cid, pl.ds(r0, ROWS_PER_SUB)])

    @pl.when(sid == NS - 1)
    def _():
        pltpu.sync_copy(acc_sh.at[pl.ds(TAIL_START, TAIL_ROWS)],
                        out_hbm.at[cid, pl.ds(TAIL_START, TAIL_ROWS)])


_agg_call = functools.partial(
    pl.kernel,
    out_type=jax.ShapeDtypeStruct((NC, N, D), jnp.float32),
    mesh=_SC_MESH,
    scratch_types=[
        pltpu.VMEM((CHUNK,), jnp.int32),
        pltpu.VMEM((CHUNK,), jnp.int32),
        pltpu.VMEM((CHUNK, D), jnp.float32),
        pltpu.VMEM_SHARED((N, D), jnp.float32),
        pltpu.SemaphoreType.DMA,
    ],
)(_agg_body)


def _tc_body(h_ref, p_ref, batch_ref, w1_ref, b1_ref, w2_ref, b2_ref,
             g_ref, be_ref, x_ref, pool_ref):
    h = h_ref[...]
    m = h + p_ref[0] + p_ref[1]
    t = jnp.maximum(
        lax.dot(m, w1_ref[...], preferred_element_type=jnp.float32)
        + b1_ref[...], 0.0)
    t = lax.dot(t, w2_ref[...], preferred_element_type=jnp.float32) + b2_ref[...]
    h1 = jnp.maximum(t, 0.0)
    mu = jnp.mean(h1, axis=0, keepdims=True)
    var = jnp.mean((h1 - mu) ** 2, axis=0, keepdims=True)
    xn = (h1 - mu) * lax.rsqrt(var + 1e-5) * g_ref[...] + be_ref[...]
    x_ref[...] = xn
    seg = lax.broadcasted_iota(jnp.int32, (G, N), 0)
    onehot_t = (batch_ref[...] == seg).astype(jnp.float32)
    sums = lax.dot(onehot_t, xn, preferred_element_type=jnp.float32)
    counts = jnp.sum(onehot_t, axis=1, keepdims=True)
    pool_ref[...] = sums / jnp.maximum(counts, 1.0)


_tc_call = pl.pallas_call(
    _tc_body,
    out_shape=[
        jax.ShapeDtypeStruct((N, D), jnp.float32),
        jax.ShapeDtypeStruct((G, D), jnp.float32),
    ],
)


def kernel(x, edge_index, batch,
           W1_0, b1_0, W2_0, b2_0, gamma_0, beta_0,
           W1_1, b1_1, W2_1, b2_1, gamma_1, beta_1,
           W1_2, b1_2, W2_2, b2_2, gamma_2, beta_2):
    src = edge_index[0]
    dst = edge_index[1]
    batch_row = batch.reshape(1, N)
    zeros = jnp.zeros((N, D), jnp.float32)
    params = [
        (W1_0, b1_0, W2_0, b2_0, gamma_0, beta_0),
        (W1_1, b1_1, W2_1, b2_1, gamma_1, beta_1),
        (W1_2, b1_2, W2_2, b2_2, gamma_2, beta_2),
    ]
    h = x
    xs, pools = [], []
    for (w1, b1, w2, b2, g, be) in params:
        parts = _agg_call(src, dst, h, zeros)
        x_l, pool_l = _tc_call(h, parts, batch_row,
                               w1, b1.reshape(1, D), w2, b2.reshape(1, D),
                               g.reshape(1, D), be.reshape(1, D))
        xs.append(x_l)
        pools.append(pool_l)
        h = x_l
    return jnp.concatenate(pools, axis=1), jnp.concatenate(xs, axis=1)


# speedup vs baseline: 7.3963x; 1.5957x over previous
import functools

import jax
import jax.numpy as jnp
from jax import lax
from jax.experimental import pallas as pl
from jax.experimental.pallas import tpu as pltpu
from jax.experimental.pallas import tpu_sc as plsc

N = 10000
E = 320000
D = 128
G = 128

NC = 2
NS = 16
NW = NC * NS

CHUNK = 40
EPW = E // NW
NCHUNK = EPW // CHUNK
KPIPE = 5
NBODY = NCHUNK // KPIPE
ROWS_PER_SUB = 624
TAIL_START = NS * ROWS_PER_SUB
TAIL_ROWS = N - TAIL_START

_SC_MESH = plsc.VectorSubcoreMesh(core_axis_name="c", subcore_axis_name="s")


def _agg_body(src_hbm, dst_hbm, h_hbm, zero_hbm, out_hbm,
              src_v, dst_v, rows_v, acc_sh, isem,
              gsem0, gsem1, gsem2, gsem3, gsem4):
    gsems = [gsem0, gsem1, gsem2, gsem3, gsem4]
    cid = lax.axis_index("c")
    sid = lax.axis_index("s")
    wid = sid * NC + cid
    r0 = sid * ROWS_PER_SUB
    pltpu.sync_copy(zero_hbm.at[pl.ds(r0, ROWS_PER_SUB)],
                    acc_sh.at[pl.ds(r0, ROWS_PER_SUB)])

    @pl.when(sid == NS - 1)
    def _():
        pltpu.sync_copy(zero_hbm.at[pl.ds(TAIL_START, TAIL_ROWS)],
                        acc_sh.at[pl.ds(TAIL_START, TAIL_ROWS)])

    plsc.subcore_barrier()

    def body(g, carry):
        base0 = wid * EPW + g * (KPIPE * CHUNK)
        idx_cps = []
        for j in range(KPIPE):
            b = base0 + j * CHUNK
            idx_cps.append(
                pltpu.async_copy(src_hbm.at[pl.ds(b, CHUNK)], src_v.at[j], isem))
            idx_cps.append(
                pltpu.async_copy(dst_hbm.at[pl.ds(b, CHUNK)], dst_v.at[j], isem))
        for cp in idx_cps:
            cp.wait()
        g_cps = [
            pltpu.async_copy(h_hbm.at[src_v.at[j]], rows_v.at[j], gsems[j])
            for j in range(KPIPE)
        ]
        for j in range(KPIPE):
            g_cps[j].wait()
            pltpu.sync_copy(rows_v.at[j], acc_sh.at[dst_v.at[j]], add=True)
        return carry

    lax.fori_loop(0, NBODY, body, 0)
    plsc.subcore_barrier()
    pltpu.sync_copy(acc_sh.at[pl.ds(r0, ROWS_PER_SUB)],
                    out_hbm.at[cid, pl.ds(r0, ROWS_PER_SUB)])

    @pl.when(sid == NS - 1)
    def _():
        pltpu.sync_copy(acc_sh.at[pl.ds(TAIL_START, TAIL_ROWS)],
                        out_hbm.at[cid, pl.ds(TAIL_START, TAIL_ROWS)])


_agg_call = functools.partial(
    pl.kernel,
    out_type=jax.ShapeDtypeStruct((NC, N, D), jnp.float32),
    mesh=_SC_MESH,
    scratch_types=[
        pltpu.VMEM((KPIPE, CHUNK), jnp.int32),
        pltpu.VMEM((KPIPE, CHUNK), jnp.int32),
        pltpu.VMEM((KPIPE, CHUNK, D), jnp.float32),
        pltpu.VMEM_SHARED((N, D), jnp.float32),
        pltpu.SemaphoreType.DMA,
        pltpu.SemaphoreType.DMA,
        pltpu.SemaphoreType.DMA,
        pltpu.SemaphoreType.DMA,
        pltpu.SemaphoreType.DMA,
        pltpu.SemaphoreType.DMA,
    ],
)(_agg_body)


def _tc_body(h_ref, p_ref, batch_ref, w1_ref, b1_ref, w2_ref, b2_ref,
             g_ref, be_ref, x_ref, pool_ref):
    h = h_ref[...]
    m = h + p_ref[0] + p_ref[1]
    t = jnp.maximum(
        lax.dot(m, w1_ref[...], preferred_element_type=jnp.float32)
        + b1_ref[...], 0.0)
    t = lax.dot(t, w2_ref[...], preferred_element_type=jnp.float32) + b2_ref[...]
    h1 = jnp.maximum(t, 0.0)
    mu = jnp.mean(h1, axis=0, keepdims=True)
    var = jnp.mean((h1 - mu) ** 2, axis=0, keepdims=True)
    xn = (h1 - mu) * lax.rsqrt(var + 1e-5) * g_ref[...] + be_ref[...]
    x_ref[...] = xn
    seg = lax.broadcasted_iota(jnp.int32, (G, N), 0)
    onehot_t = (batch_ref[...] == seg).astype(jnp.float32)
    sums = lax.dot(onehot_t, xn, preferred_element_type=jnp.float32)
    counts = jnp.sum(onehot_t, axis=1, keepdims=True)
    pool_ref[...] = sums / jnp.maximum(counts, 1.0)


_tc_call = pl.pallas_call(
    _tc_body,
    out_shape=[
        jax.ShapeDtypeStruct((N, D), jnp.float32),
        jax.ShapeDtypeStruct((G, D), jnp.float32),
    ],
)


def kernel(x, edge_index, batch,
           W1_0, b1_0, W2_0, b2_0, gamma_0, beta_0,
           W1_1, b1_1, W2_1, b2_1, gamma_1, beta_1,
           W1_2, b1_2, W2_2, b2_2, gamma_2, beta_2):
    src = edge_index[0]
    dst = edge_index[1]
    batch_row = batch.reshape(1, N)
    zeros = jnp.zeros((N, D), jnp.float32)
    params = [
        (W1_0, b1_0, W2_0, b2_0, gamma_0, beta_0),
        (W1_1, b1_1, W2_1, b2_1, gamma_1, beta_1),
        (W1_2, b1_2, W2_2, b2_2, gamma_2, beta_2),
    ]
    h = x
    xs, pools = [], []
    for (w1, b1, w2, b2, g, be) in params:
        parts = _agg_call(src, dst, h, zeros)
        x_l, pool_l = _tc_call(h, parts, batch_row,
                               w1, b1.reshape(1, D), w2, b2.reshape(1, D),
                               g.reshape(1, D), be.reshape(1, D))
        xs.append(x_l)
        pools.append(pool_l)
        h = x_l
    return jnp.concatenate(pools, axis=1), jnp.concatenate(xs, axis=1)


# speedup vs baseline: 7.8294x; 1.0586x over previous
import functools

import jax
import jax.numpy as jnp
from jax import lax
from jax.experimental import pallas as pl
from jax.experimental.pallas import tpu as pltpu
from jax.experimental.pallas import tpu_sc as plsc

N = 10000
E = 320000
D = 128
G = 128

NC = 2
NS = 16
NW = NC * NS

CHUNK = 40
EPW = E // NW
NCHUNK = EPW // CHUNK
KPIPE = 5
NBODY = NCHUNK // KPIPE
ROWS_PER_SUB = 624
TAIL_START = NS * ROWS_PER_SUB
TAIL_ROWS = N - TAIL_START

_SC_MESH = plsc.VectorSubcoreMesh(core_axis_name="c", subcore_axis_name="s")


def _agg_body(src_hbm, dst_hbm, h_hbm, zero_hbm, out_hbm,
              src_v, dst_v, rows_v, acc_sh, isem, ssem,
              gsem0, gsem1, gsem2, gsem3, gsem4):
    gsems = [gsem0, gsem1, gsem2, gsem3, gsem4]
    cid = lax.axis_index("c")
    sid = lax.axis_index("s")
    wid = sid * NC + cid
    r0 = sid * ROWS_PER_SUB
    pltpu.sync_copy(zero_hbm.at[pl.ds(r0, ROWS_PER_SUB)],
                    acc_sh.at[pl.ds(r0, ROWS_PER_SUB)])

    @pl.when(sid == NS - 1)
    def _():
        pltpu.sync_copy(zero_hbm.at[pl.ds(TAIL_START, TAIL_ROWS)],
                        acc_sh.at[pl.ds(TAIL_START, TAIL_ROWS)])

    plsc.subcore_barrier()

    def body(g, carry):
        base0 = wid * EPW + g * (KPIPE * CHUNK)
        idx_cps = []
        for j in range(KPIPE):
            b = base0 + j * CHUNK
            idx_cps.append(
                pltpu.async_copy(src_hbm.at[pl.ds(b, CHUNK)], src_v.at[j], isem))
            idx_cps.append(
                pltpu.async_copy(dst_hbm.at[pl.ds(b, CHUNK)], dst_v.at[j], isem))
        for cp in idx_cps:
            cp.wait()
        g_cps = [
            pltpu.async_copy(h_hbm.at[src_v.at[j]], rows_v.at[j], gsems[j])
            for j in range(KPIPE)
        ]
        s_cps = []
        for j in range(KPIPE):
            g_cps[j].wait()
            s_cps.append(
                pltpu.async_copy(rows_v.at[j], acc_sh.at[dst_v.at[j]], ssem,
                                 add=True))
        for cp in s_cps:
            cp.wait()
        return carry

    lax.fori_loop(0, NBODY, body, 0)
    plsc.subcore_barrier()
    pltpu.sync_copy(acc_sh.at[pl.ds(r0, ROWS_PER_SUB)],
                    out_hbm.at[cid, pl.ds(r0, ROWS_PER_SUB)])

    @pl.when(sid == NS - 1)
    def _():
        pltpu.sync_copy(acc_sh.at[pl.ds(TAIL_START, TAIL_ROWS)],
                        out_hbm.at[cid, pl.ds(TAIL_START, TAIL_ROWS)])


_agg_call = functools.partial(
    pl.kernel,
    out_type=jax.ShapeDtypeStruct((NC, N, D), jnp.float32),
    mesh=_SC_MESH,
    scratch_types=[
        pltpu.VMEM((KPIPE, CHUNK), jnp.int32),
        pltpu.VMEM((KPIPE, CHUNK), jnp.int32),
        pltpu.VMEM((KPIPE, CHUNK, D), jnp.float32),
        pltpu.VMEM_SHARED((N, D), jnp.float32),
        pltpu.SemaphoreType.DMA,
        pltpu.SemaphoreType.DMA,
        pltpu.SemaphoreType.DMA,
        pltpu.SemaphoreType.DMA,
        pltpu.SemaphoreType.DMA,
        pltpu.SemaphoreType.DMA,
        pltpu.SemaphoreType.DMA,
    ],
)(_agg_body)


def _tc_body(h_ref, p_ref, batch_ref, w1_ref, b1_ref, w2_ref, b2_ref,
             g_ref, be_ref, x_ref, pool_ref):
    h = h_ref[...]
    m = h + p_ref[0] + p_ref[1]
    t = jnp.maximum(
        lax.dot(m, w1_ref[...], preferred_element_type=jnp.float32)
        + b1_ref[...], 0.0)
    t = lax.dot(t, w2_ref[...], preferred_element_type=jnp.float32) + b2_ref[...]
    h1 = jnp.maximum(t, 0.0)
    mu = jnp.mean(h1, axis=0, keepdims=True)
    var = jnp.mean((h1 - mu) ** 2, axis=0, keepdims=True)
    xn = (h1 - mu) * lax.rsqrt(var + 1e-5) * g_ref[...] + be_ref[...]
    x_ref[...] = xn
    seg = lax.broadcasted_iota(jnp.int32, (G, N), 0)
    onehot_t = (batch_ref[...] == seg).astype(jnp.float32)
    sums = lax.dot(onehot_t, xn, preferred_element_type=jnp.float32)
    counts = jnp.sum(onehot_t, axis=1, keepdims=True)
    pool_ref[...] = sums / jnp.maximum(counts, 1.0)


_tc_call = pl.pallas_call(
    _tc_body,
    out_shape=[
        jax.ShapeDtypeStruct((N, D), jnp.float32),
        jax.ShapeDtypeStruct((G, D), jnp.float32),
    ],
)


def kernel(x, edge_index, batch,
           W1_0, b1_0, W2_0, b2_0, gamma_0, beta_0,
           W1_1, b1_1, W2_1, b2_1, gamma_1, beta_1,
           W1_2, b1_2, W2_2, b2_2, gamma_2, beta_2):
    src = edge_index[0]
    dst = edge_index[1]
    batch_row = batch.reshape(1, N)
    zeros = jnp.zeros((N, D), jnp.float32)
    params = [
        (W1_0, b1_0, W2_0, b2_0, gamma_0, beta_0),
        (W1_1, b1_1, W2_1, b2_1, gamma_1, beta_1),
        (W1_2, b1_2, W2_2, b2_2, gamma_2, beta_2),
    ]
    h = x
    xs, pools = [], []
    for (w1, b1, w2, b2, g, be) in params:
        parts = _agg_call(src, dst, h, zeros)
        x_l, pool_l = _tc_call(h, parts, batch_row,
                               w1, b1.reshape(1, D), w2, b2.reshape(1, D),
                               g.reshape(1, D), be.reshape(1, D))
        xs.append(x_l)
        pools.append(pool_l)
        h = x_l
    return jnp.concatenate(pools, axis=1), jnp.concatenate(xs, axis=1)


# speedup vs baseline: 9.0535x; 1.1563x over previous
import functools

import jax
import jax.numpy as jnp
from jax import lax
from jax.experimental import pallas as pl
from jax.experimental.pallas import tpu as pltpu
from jax.experimental.pallas import tpu_sc as plsc

N = 10000
E = 320000
D = 128
G = 128

NC = 2
NS = 16
NW = NC * NS

CHUNK = 40
EPW = E // NW
NCHUNK = EPW // CHUNK
KPIPE = 9
NBODY = NCHUNK // KPIPE
NTAIL = NCHUNK - NBODY * KPIPE
ROWS_PER_SUB = 624
TAIL_START = NS * ROWS_PER_SUB
TAIL_ROWS = N - TAIL_START

_SC_MESH = plsc.VectorSubcoreMesh(core_axis_name="c", subcore_axis_name="s")


def _agg_body(src_hbm, dst_hbm, h_hbm, zero_hbm, out_hbm,
              src_v, dst_v, rows_v, acc_sh, isem, ssem, *gsems):
    cid = lax.axis_index("c")
    sid = lax.axis_index("s")
    wid = sid * NC + cid
    r0 = sid * ROWS_PER_SUB
    pltpu.sync_copy(zero_hbm.at[pl.ds(r0, ROWS_PER_SUB)],
                    acc_sh.at[pl.ds(r0, ROWS_PER_SUB)])

    @pl.when(sid == NS - 1)
    def _():
        pltpu.sync_copy(zero_hbm.at[pl.ds(TAIL_START, TAIL_ROWS)],
                        acc_sh.at[pl.ds(TAIL_START, TAIL_ROWS)])

    plsc.subcore_barrier()

    def run_chunks(base0, k):
        idx_cps = []
        for j in range(k):
            b = base0 + j * CHUNK
            idx_cps.append(
                pltpu.async_copy(src_hbm.at[pl.ds(b, CHUNK)], src_v.at[j], isem))
            idx_cps.append(
                pltpu.async_copy(dst_hbm.at[pl.ds(b, CHUNK)], dst_v.at[j], isem))
        for cp in idx_cps:
            cp.wait()
        g_cps = [
            pltpu.async_copy(h_hbm.at[src_v.at[j]], rows_v.at[j], gsems[j])
            for j in range(k)
        ]
        s_cps = []
        for j in range(k):
            g_cps[j].wait()
            s_cps.append(
                pltpu.async_copy(rows_v.at[j], acc_sh.at[dst_v.at[j]], ssem,
                                 add=True))
        for cp in s_cps:
            cp.wait()

    def body(g, carry):
        run_chunks(wid * EPW + g * (KPIPE * CHUNK), KPIPE)
        return carry

    lax.fori_loop(0, NBODY, body, 0)
    if NTAIL:
        run_chunks(wid * EPW + NBODY * KPIPE * CHUNK, NTAIL)
    plsc.subcore_barrier()
    pltpu.sync_copy(acc_sh.at[pl.ds(r0, ROWS_PER_SUB)],
                    out_hbm.at[cid, pl.ds(r0, ROWS_PER_SUB)])

    @pl.when(sid == NS - 1)
    def _():
        pltpu.sync_copy(acc_sh.at[pl.ds(TAIL_START, TAIL_ROWS)],
                        out_hbm.at[cid, pl.ds(TAIL_START, TAIL_ROWS)])


_agg_call = functools.partial(
    pl.kernel,
    out_type=jax.ShapeDtypeStruct((NC, N, D), jnp.float32),
    mesh=_SC_MESH,
    scratch_types=[
        pltpu.VMEM((KPIPE, CHUNK), jnp.int32),
        pltpu.VMEM((KPIPE, CHUNK), jnp.int32),
        pltpu.VMEM((KPIPE, CHUNK, D), jnp.float32),
        pltpu.VMEM_SHARED((N, D), jnp.float32),
    ] + [pltpu.SemaphoreType.DMA] * (2 + KPIPE),
)(_agg_body)


def _tc_body(h_ref, p_ref, batch_ref, w1_ref, b1_ref, w2_ref, b2_ref,
             g_ref, be_ref, x_ref, pool_ref):
    h = h_ref[...]
    m = h + p_ref[0] + p_ref[1]
    t = jnp.maximum(
        lax.dot(m, w1_ref[...], preferred_element_type=jnp.float32)
        + b1_ref[...], 0.0)
    t = lax.dot(t, w2_ref[...], preferred_element_type=jnp.float32) + b2_ref[...]
    h1 = jnp.maximum(t, 0.0)
    mu = jnp.mean(h1, axis=0, keepdims=True)
    var = jnp.mean((h1 - mu) ** 2, axis=0, keepdims=True)
    xn = (h1 - mu) * lax.rsqrt(var + 1e-5) * g_ref[...] + be_ref[...]
    x_ref[...] = xn
    seg = lax.broadcasted_iota(jnp.int32, (G, N), 0)
    onehot_t = (batch_ref[...] == seg).astype(jnp.float32)
    sums = lax.dot(onehot_t, xn, preferred_element_type=jnp.float32)
    counts = jnp.sum(onehot_t, axis=1, keepdims=True)
    pool_ref[...] = sums / jnp.maximum(counts, 1.0)


_tc_call = pl.pallas_call(
    _tc_body,
    out_shape=[
        jax.ShapeDtypeStruct((N, D), jnp.float32),
        jax.ShapeDtypeStruct((G, D), jnp.float32),
    ],
)


def kernel(x, edge_index, batch,
           W1_0, b1_0, W2_0, b2_0, gamma_0, beta_0,
           W1_1, b1_1, W2_1, b2_1, gamma_1, beta_1,
           W1_2, b1_2, W2_2, b2_2, gamma_2, beta_2):
    src = edge_index[0]
    dst = edge_index[1]
    batch_row = batch.reshape(1, N)
    zeros = jnp.zeros((N, D), jnp.float32)
    params = [
        (W1_0, b1_0, W2_0, b2_0, gamma_0, beta_0),
        (W1_1, b1_1, W2_1, b2_1, gamma_1, beta_1),
        (W1_2, b1_2, W2_2, b2_2, gamma_2, beta_2),
    ]
    h = x
    xs, pools = [], []
    for (w1, b1, w2, b2, g, be) in params:
        parts = _agg_call(src, dst, h, zeros)
        x_l, pool_l = _tc_call(h, parts, batch_row,
                               w1, b1.reshape(1, D), w2, b2.reshape(1, D),
                               g.reshape(1, D), be.reshape(1, D))
        xs.append(x_l)
        pools.append(pool_l)
        h = x_l
    return jnp.concatenate(pools, axis=1), jnp.concatenate(xs, axis=1)


# speedup vs baseline: 9.1441x; 1.0100x over previous
import functools

import jax
import jax.numpy as jnp
from jax import lax
from jax.experimental import pallas as pl
from jax.experimental.pallas import tpu as pltpu
from jax.experimental.pallas import tpu_sc as plsc

N = 10000
E = 320000
D = 128
G = 128

NC = 2
NS = 16
NW = NC * NS

CHUNK = 40
EPW = E // NW
NCHUNK = EPW // CHUNK
KPIPE = 9
NBODY = NCHUNK // KPIPE
NTAIL = NCHUNK - NBODY * KPIPE
ROWS_PER_SUB = 624
TAIL_START = NS * ROWS_PER_SUB
TAIL_ROWS = N - TAIL_START

_SC_MESH = plsc.VectorSubcoreMesh(core_axis_name="c", subcore_axis_name="s")


def _agg_body(src_hbm, dst_hbm, h_hbm, zero_hbm, out_hbm,
              src_v, dst_v, rows_v, acc_sh, isem, ssem, *gsems):
    cid = lax.axis_index("c")
    sid = lax.axis_index("s")
    wid = sid * NC + cid
    r0 = sid * ROWS_PER_SUB
    pltpu.sync_copy(zero_hbm.at[pl.ds(r0, ROWS_PER_SUB)],
                    acc_sh.at[pl.ds(r0, ROWS_PER_SUB)])

    @pl.when(sid == NS - 1)
    def _():
        pltpu.sync_copy(zero_hbm.at[pl.ds(TAIL_START, TAIL_ROWS)],
                        acc_sh.at[pl.ds(TAIL_START, TAIL_ROWS)])

    plsc.subcore_barrier()

    def run_chunks(base0, k):
        idx_cps = []
        for j in range(k):
            b = base0 + j * CHUNK
            idx_cps.append(
                pltpu.async_copy(src_hbm.at[pl.ds(b, CHUNK)], src_v.at[j], isem))
            idx_cps.append(
                pltpu.async_copy(dst_hbm.at[pl.ds(b, CHUNK)], dst_v.at[j], isem))
        for cp in idx_cps:
            cp.wait()
        g_cps = [
            pltpu.async_copy(h_hbm.at[src_v.at[j]], rows_v.at[j], gsems[j])
            for j in range(k)
        ]
        s_cps = []
        for j in range(k):
            g_cps[j].wait()
            s_cps.append(
                pltpu.async_copy(rows_v.at[j], acc_sh.at[dst_v.at[j]], ssem,
                                 add=True))
        for cp in s_cps:
            cp.wait()

    def body(g, carry):
        run_chunks(wid * EPW + g * (KPIPE * CHUNK), KPIPE)
        return carry

    lax.fori_loop(0, NBODY, body, 0)
    if NTAIL:
        run_chunks(wid * EPW + NBODY * KPIPE * CHUNK, NTAIL)
    plsc.subcore_barrier()
    pltpu.sync_copy(acc_sh.at[pl.ds(r0, ROWS_PER_SUB)],
                    out_hbm.at[cid, pl.ds(r0, ROWS_PER_SUB)])

    @pl.when(sid == NS - 1)
    def _():
        pltpu.sync_copy(acc_sh.at[pl.ds(TAIL_START, TAIL_ROWS)],
                        out_hbm.at[cid, pl.ds(TAIL_START, TAIL_ROWS)])


_agg_call = functools.partial(
    pl.kernel,
    out_type=jax.ShapeDtypeStruct((NC, N, D), jnp.float32),
    mesh=_SC_MESH,
    scratch_types=[
        pltpu.VMEM((KPIPE, CHUNK), jnp.int32),
        pltpu.VMEM((KPIPE, CHUNK), jnp.int32),
        pltpu.VMEM((KPIPE, CHUNK, D), jnp.float32),
        pltpu.VMEM_SHARED((N, D), jnp.float32),
    ] + [pltpu.SemaphoreType.DMA] * (2 + KPIPE),
)(_agg_body)


def _mlp_bn(h, p_ref, w1_ref, b1_ref, w2_ref, b2_ref, g_ref, be_ref):
    m = h + p_ref[0] + p_ref[1]
    t = jnp.maximum(
        lax.dot(m, w1_ref[...], preferred_element_type=jnp.float32)
        + b1_ref[...], 0.0)
    t = lax.dot(t, w2_ref[...], preferred_element_type=jnp.float32) + b2_ref[...]
    h1 = jnp.maximum(t, 0.0)
    mu = jnp.mean(h1, axis=0, keepdims=True)
    var = jnp.mean((h1 - mu) ** 2, axis=0, keepdims=True)
    return (h1 - mu) * lax.rsqrt(var + 1e-5) * g_ref[...] + be_ref[...]


def _tc_body(h_ref, p_ref, w1_ref, b1_ref, w2_ref, b2_ref,
             g_ref, be_ref, x_ref):
    x_ref[...] = _mlp_bn(h_ref[...], p_ref, w1_ref, b1_ref, w2_ref, b2_ref,
                         g_ref, be_ref)


_tc_call = pl.pallas_call(
    _tc_body,
    out_shape=jax.ShapeDtypeStruct((N, D), jnp.float32),
)


def _tc_final_body(h_ref, p_ref, batch_ref, w1_ref, b1_ref, w2_ref, b2_ref,
                   g_ref, be_ref, x0_ref, x1_ref, xcat_ref, pool_ref):
    x2 = _mlp_bn(h_ref[...], p_ref, w1_ref, b1_ref, w2_ref, b2_ref,
                 g_ref, be_ref)
    xcat_ref[:, 0:D] = x0_ref[...]
    xcat_ref[:, D:2 * D] = x1_ref[...]
    xcat_ref[:, 2 * D:3 * D] = x2
    seg = lax.broadcasted_iota(jnp.int32, (G, N), 0)
    onehot_t = (batch_ref[...] == seg).astype(jnp.float32)
    counts = jnp.maximum(jnp.sum(onehot_t, axis=1, keepdims=True), 1.0)
    xcat = jnp.concatenate([x0_ref[...], x1_ref[...], x2], axis=1)
    sums = lax.dot(onehot_t, xcat, preferred_element_type=jnp.float32)
    pool_ref[...] = sums / counts


_tc_final_call = pl.pallas_call(
    _tc_final_body,
    out_shape=[
        jax.ShapeDtypeStruct((N, 3 * D), jnp.float32),
        jax.ShapeDtypeStruct((G, 3 * D), jnp.float32),
    ],
)


def kernel(x, edge_index, batch,
           W1_0, b1_0, W2_0, b2_0, gamma_0, beta_0,
           W1_1, b1_1, W2_1, b2_1, gamma_1, beta_1,
           W1_2, b1_2, W2_2, b2_2, gamma_2, beta_2):
    src = edge_index[0]
    dst = edge_index[1]
    batch_row = batch.reshape(1, N)
    zeros = jnp.zeros((N, D), jnp.float32)
    params = [
        (W1_0, b1_0, W2_0, b2_0, gamma_0, beta_0),
        (W1_1, b1_1, W2_1, b2_1, gamma_1, beta_1),
        (W1_2, b1_2, W2_2, b2_2, gamma_2, beta_2),
    ]
    h = x
    xs = []
    for (w1, b1, w2, b2, g, be) in params[:2]:
        parts = _agg_call(src, dst, h, zeros)
        h = _tc_call(h, parts,
                     w1, b1.reshape(1, D), w2, b2.reshape(1, D),
                     g.reshape(1, D), be.reshape(1, D))
        xs.append(h)
    (w1, b1, w2, b2, g, be) = params[2]
    parts = _agg_call(src, dst, h, zeros)
    xcat, poolcat = _tc_final_call(h, parts, batch_row,
                                   w1, b1.reshape(1, D), w2, b2.reshape(1, D),
                                   g.reshape(1, D), be.reshape(1, D),
                                   xs[0], xs[1])
    return poolcat, xcat


# speedup vs baseline: 9.2408x; 1.0106x over previous
import functools

import jax
import jax.numpy as jnp
from jax import lax
from jax.experimental import pallas as pl
from jax.experimental.pallas import tpu as pltpu
from jax.experimental.pallas import tpu_sc as plsc

N = 10000
E = 320000
D = 128
G = 128

NC = 2
NS = 16
NW = NC * NS

CHUNK = 40
EPW = E // NW
NCHUNK = EPW // CHUNK
KPIPE = 9
NBODY = NCHUNK // KPIPE
NTAIL = NCHUNK - NBODY * KPIPE
ROWS_PER_SUB = 624
TAIL_START = NS * ROWS_PER_SUB
TAIL_ROWS = N - TAIL_START

_SC_MESH = plsc.VectorSubcoreMesh(core_axis_name="c", subcore_axis_name="s")


def _agg_body(src_hbm, dst_hbm, h_hbm, zero_hbm, out_hbm,
              src_v, dst_v, rows_v, acc_sh, isem, ssem, zsem, *gsems):
    cid = lax.axis_index("c")
    sid = lax.axis_index("s")
    wid = sid * NC + cid
    r0 = sid * ROWS_PER_SUB

    def fire_chunks(base0, k):
        idx_cps = []
        for j in range(k):
            b = base0 + j * CHUNK
            idx_cps.append(
                pltpu.async_copy(src_hbm.at[pl.ds(b, CHUNK)], src_v.at[j], isem))
            idx_cps.append(
                pltpu.async_copy(dst_hbm.at[pl.ds(b, CHUNK)], dst_v.at[j], isem))
        for cp in idx_cps:
            cp.wait()
        return [
            pltpu.async_copy(h_hbm.at[src_v.at[j]], rows_v.at[j], gsems[j])
            for j in range(k)
        ]

    def drain_chunks(g_cps, k):
        s_cps = []
        for j in range(k):
            g_cps[j].wait()
            s_cps.append(
                pltpu.async_copy(rows_v.at[j], acc_sh.at[dst_v.at[j]], ssem,
                                 add=True))
        for cp in s_cps:
            cp.wait()

    z_cp = pltpu.async_copy(zero_hbm.at[pl.ds(r0, ROWS_PER_SUB)],
                            acc_sh.at[pl.ds(r0, ROWS_PER_SUB)], zsem)

    @pl.when(sid == NS - 1)
    def _():
        pltpu.async_copy(zero_hbm.at[pl.ds(TAIL_START, TAIL_ROWS)],
                         acc_sh.at[pl.ds(TAIL_START, TAIL_ROWS)], zsem).wait()

    cps0 = fire_chunks(wid * EPW, KPIPE)
    z_cp.wait()
    plsc.subcore_barrier()
    drain_chunks(cps0, KPIPE)

    def body(g, carry):
        drain_chunks(fire_chunks(wid * EPW + g * (KPIPE * CHUNK), KPIPE), KPIPE)
        return carry

    lax.fori_loop(1, NBODY, body, 0)
    if NTAIL:
        drain_chunks(fire_chunks(wid * EPW + NBODY * KPIPE * CHUNK, NTAIL),
                     NTAIL)
    plsc.subcore_barrier()
    pltpu.sync_copy(acc_sh.at[pl.ds(r0, ROWS_PER_SUB)],
                    out_hbm.at[cid, pl.ds(r0, ROWS_PER_SUB)])

    @pl.when(sid == NS - 1)
    def _():
        pltpu.sync_copy(acc_sh.at[pl.ds(TAIL_START, TAIL_ROWS)],
                        out_hbm.at[cid, pl.ds(TAIL_START, TAIL_ROWS)])


_agg_call = functools.partial(
    pl.kernel,
    out_type=jax.ShapeDtypeStruct((NC, N, D), jnp.float32),
    mesh=_SC_MESH,
    scratch_types=[
        pltpu.VMEM((KPIPE, CHUNK), jnp.int32),
        pltpu.VMEM((KPIPE, CHUNK), jnp.int32),
        pltpu.VMEM((KPIPE, CHUNK, D), jnp.float32),
        pltpu.VMEM_SHARED((N, D), jnp.float32),
    ] + [pltpu.SemaphoreType.DMA] * (3 + KPIPE),
)(_agg_body)


def _mlp_bn(h, p_ref, w1_ref, b1_ref, w2_ref, b2_ref, g_ref, be_ref):
    m = h + p_ref[0] + p_ref[1]
    t = jnp.maximum(
        lax.dot(m, w1_ref[...], preferred_element_type=jnp.float32)
        + b1_ref[...], 0.0)
    t = lax.dot(t, w2_ref[...], preferred_element_type=jnp.float32) + b2_ref[...]
    h1 = jnp.maximum(t, 0.0)
    mu = jnp.mean(h1, axis=0, keepdims=True)
    var = jnp.mean((h1 - mu) ** 2, axis=0, keepdims=True)
    return (h1 - mu) * lax.rsqrt(var + 1e-5) * g_ref[...] + be_ref[...]


def _tc_body(h_ref, p_ref, w1_ref, b1_ref, w2_ref, b2_ref,
             g_ref, be_ref, x_ref):
    x_ref[...] = _mlp_bn(h_ref[...], p_ref, w1_ref, b1_ref, w2_ref, b2_ref,
                         g_ref, be_ref)


_tc_call = pl.pallas_call(
    _tc_body,
    out_shape=jax.ShapeDtypeStruct((N, D), jnp.float32),
)


def _tc_final_body(h_ref, p_ref, batch_ref, w1_ref, b1_ref, w2_ref, b2_ref,
                   g_ref, be_ref, x0_ref, x1_ref, xcat_ref, pool_ref):
    x2 = _mlp_bn(h_ref[...], p_ref, w1_ref, b1_ref, w2_ref, b2_ref,
                 g_ref, be_ref)
    xcat_ref[:, 0:D] = x0_ref[...]
    xcat_ref[:, D:2 * D] = x1_ref[...]
    xcat_ref[:, 2 * D:3 * D] = x2
    seg = lax.broadcasted_iota(jnp.int32, (G, N), 0)
    onehot_t = (batch_ref[...] == seg).astype(jnp.float32)
    counts = jnp.maximum(jnp.sum(onehot_t, axis=1, keepdims=True), 1.0)
    xcat = jnp.concatenate([x0_ref[...], x1_ref[...], x2], axis=1)
    sums = lax.dot(onehot_t, xcat, preferred_element_type=jnp.float32)
    pool_ref[...] = sums / counts


_tc_final_call = pl.pallas_call(
    _tc_final_body,
    out_shape=[
        jax.ShapeDtypeStruct((N, 3 * D), jnp.float32),
        jax.ShapeDtypeStruct((G, 3 * D), jnp.float32),
    ],
)


def kernel(x, edge_index, batch,
           W1_0, b1_0, W2_0, b2_0, gamma_0, beta_0,
           W1_1, b1_1, W2_1, b2_1, gamma_1, beta_1,
           W1_2, b1_2, W2_2, b2_2, gamma_2, beta_2):
    src = edge_index[0]
    dst = edge_index[1]
    batch_row = batch.reshape(1, N)
    zeros = jnp.zeros((N, D), jnp.float32)
    params = [
        (W1_0, b1_0, W2_0, b2_0, gamma_0, beta_0),
        (W1_1, b1_1, W2_1, b2_1, gamma_1, beta_1),
        (W1_2, b1_2, W2_2, b2_2, gamma_2, beta_2),
    ]
    h = x
    xs = []
    for (w1, b1, w2, b2, g, be) in params[:2]:
        parts = _agg_call(src, dst, h, zeros)
        h = _tc_call(h, parts,
                     w1, b1.reshape(1, D), w2, b2.reshape(1, D),
                     g.reshape(1, D), be.reshape(1, D))
        xs.append(h)
    (w1, b1, w2, b2, g, be) = params[2]
    parts = _agg_call(src, dst, h, zeros)
    xcat, poolcat = _tc_final_call(h, parts, batch_row,
                                   w1, b1.reshape(1, D), w2, b2.reshape(1, D),
                                   g.reshape(1, D), be.reshape(1, D),
                                   xs[0], xs[1])
    return poolcat, xcat
